# Initial kernel scaffold; baseline (speedup 1.0000x reference)
import jax
import jax.numpy as jnp
from jax.experimental import pallas as pl


def kernel(x, gate_W, W1, b1, W2, b2, Ws, bs, gamma, beta):
    raise NotImplementedError("write your pallas kernel here")


# speedup vs baseline: 3.6093x; 3.6093x over previous
import functools
import jax
import jax.numpy as jnp
from jax.experimental import pallas as pl
from jax.experimental.pallas import tpu as pltpu

N, D, E, H, TOPK = 8192, 768, 8, 512, 2
TB = 256


def _gelu(v):
    return 0.5 * v * (1.0 + jax.lax.erf(v * 0.7071067811865476))


def _top2(logits):
    T = logits.shape[0]
    iota = jax.lax.broadcasted_iota(jnp.int32, (T, E), 1)
    m1 = jnp.max(logits, axis=-1, keepdims=True)
    i1 = jnp.min(jnp.where(logits == m1, iota, E), axis=-1, keepdims=True)
    l2 = jnp.where(iota == i1, -jnp.inf, logits)
    m2 = jnp.max(l2, axis=-1, keepdims=True)
    i2 = jnp.min(jnp.where(l2 == m2, iota, E), axis=-1, keepdims=True)
    r = jnp.exp(m2 - m1)
    w_a = 1.0 / (1.0 + r)
    w_b = 1.0 - w_a
    return w_a, w_b, i1, i2


def _dense_body(x_ref, gw_ref, w1_ref, b1_ref, w2_ref, b2_ref, ws_ref, bs_ref,
                g_ref, be_ref, o_ref):
    xb = x_ref[...]
    logits = jnp.dot(xb, gw_ref[...], preferred_element_type=jnp.float32)
    w_a, w_b, i1, i2 = _top2(logits)
    acc = jnp.zeros((TB, D), jnp.float32)
    for e in range(E):
        h = _gelu(jnp.dot(xb, w1_ref[e], preferred_element_type=jnp.float32)
                  + b1_ref[e][None, :])
        y = jnp.dot(h, w2_ref[e], preferred_element_type=jnp.float32) + b2_ref[e][None, :]
        coef = jnp.where(i1 == e, w_a, 0.0) + jnp.where(i2 == e, w_b, 0.0)
        acc = acc + coef * y
    shared = _gelu(jnp.dot(xb, ws_ref[...], preferred_element_type=jnp.float32)
                   + bs_ref[...][None, :])
    out = acc + 0.5 * shared
    mu = jnp.mean(out, axis=-1, keepdims=True)
    d = out - mu
    var = jnp.mean(d * d, axis=-1, keepdims=True)
    o_ref[...] = d * jax.lax.rsqrt(var + 1e-5) * g_ref[...][None, :] + be_ref[...][None, :]


@jax.jit
def kernel(x, gate_W, W1, b1, W2, b2, Ws, bs, gamma, beta):
    full = lambda shape: pl.BlockSpec(shape, lambda i: (0,) * len(shape))
    return pl.pallas_call(
        _dense_body,
        grid=(N // TB,),
        in_specs=[
            pl.BlockSpec((TB, D), lambda i: (i, 0)),
            full((D, E)),
            full((E, D, H)),
            full((E, H)),
            full((E, H, D)),
            full((E, D)),
            full((D, D)),
            full((D,)),
            full((D,)),
            full((D,)),
        ],
        out_specs=pl.BlockSpec((TB, D), lambda i: (i, 0)),
        out_shape=jax.ShapeDtypeStruct((N, D), jnp.float32),
    )(x, gate_W, W1, b1, W2, b2, Ws, bs, gamma, beta)
